# baseline (speedup 1.0000x reference)
import functools

import jax
import jax.numpy as jnp
from jax.experimental import pallas as pl
from jax.experimental.pallas import tpu as pltpu

_PARAMS = pltpu.CompilerParams(dimension_semantics=("parallel",))


def _nsum(z2, H, W):
    C = z2.shape[-1]
    x3 = z2.reshape(H, W, C)
    up = jnp.concatenate([x3[1:], x3[:1]], axis=0)
    dn = jnp.concatenate([x3[-1:], x3[:-1]], axis=0)
    lf = jnp.concatenate([x3[:, 1:], x3[:, :1]], axis=1)
    rt = jnp.concatenate([x3[:, -1:], x3[:, :-1]], axis=1)
    return ((up + dn) + (lf + rt)).reshape(H * W, C)


def _sage_pre(x2, wm, b, H, W):
    xa = jnp.concatenate([x2, _nsum(x2, H, W)], axis=1)
    return jnp.dot(xa, wm, preferred_element_type=jnp.float32) + b


def _sage_post(x2, wm, b, H, W, co):
    yz = jnp.dot(x2, wm, preferred_element_type=jnp.float32)
    return yz[:, :co] + _nsum(yz[:, co:], H, W) + b


def _act(y):
    return jax.nn.relu(y).astype(jnp.bfloat16)


def _body(x_ref, w0, b0, w1, b1, w2, b2, w3, b3, w4, b4, wt, bt,
          w5, b5, w6, b6, w7, b7, o_ref, *, H, W):
    relu = jax.nn.relu
    x = x_ref[0].astype(jnp.bfloat16)
    x0 = _act(_sage_pre(x, w0[...], b0[...], H, W))
    h1 = _act(_sage_pre(x0, w1[...], b1[...], H, W))
    h2 = _act(_sage_pre(h1, w2[...], b2[...], H, W))

    C = h2.shape[-1]
    x3 = h2.reshape(H, W, C)
    mw = jnp.maximum(x3, jnp.concatenate([x3[:, 1:], x3[:, :1]], axis=1))
    mh = jnp.maximum(mw, jnp.concatenate([mw[1:], mw[:1]], axis=0))
    pe = mh.reshape(H // 2, 2, W, C)[:, 0]
    p = pltpu.einshape("a(bp)c->pabc", pe, p=2)[0]
    p = p.reshape(H * W // 4, C)

    h2d, w2d = H // 2, W // 2
    l1 = _act(_sage_pre(p, w3[...], b3[...], h2d, w2d))
    l2 = _act(_sage_pre(l1, w4[...], b4[...], h2d, w2d))

    u = jnp.dot(l2, wt[...], preferred_element_type=jnp.float32) + bt[...]
    uc = u.shape[-1] // 4
    u5 = u.reshape(h2d, w2d, 2, 2, uc).astype(jnp.bfloat16)
    u48 = pltpu.einshape("hwklc->(hk)(wl)c", u5).reshape(H * W, uc)

    cat = jnp.concatenate([h2, u48], axis=1)
    c5 = _act(_sage_post(cat, w5[...], b5[...], H, W, w5.shape[-1] // 2))
    c6 = _act(_sage_pre(c5, w6[...], b6[...], H, W))
    o_ref[0] = _sage_pre(c6, w7[...], b7[...], H, W)


def _const_spec(shape):
    nd = len(shape)
    return pl.BlockSpec(shape, lambda i: (0,) * nd)


def _tile_spec(shape):
    return pl.BlockSpec((1,) + shape[1:], lambda i: (i,) + (0,) * (len(shape) - 1))


def kernel(inputs, ws0, wn0, b0, ws1, wn1, b1, ws2, wn2, b2, ws3, wn3, b3,
           ws4, wn4, b4, ws5, wn5, b5, ws6, wn6, b6, ws7, wn7, b7, wt, bt):
    B, T, NX, NY, C = inputs.shape
    BT = B * T
    HW = NX * NY
    f32 = jnp.float32

    w0 = jnp.concatenate([ws0, wn0 * 0.25], axis=0)
    w1 = jnp.concatenate([ws1, wn1 * 0.25], axis=0)
    w2 = jnp.concatenate([ws2, wn2 * 0.25], axis=0)
    w3 = jnp.concatenate([ws3, wn3 * 0.25], axis=0)
    w4 = jnp.concatenate([ws4, wn4 * 0.25], axis=0)
    w5 = jnp.concatenate([ws5, wn5 * 0.25], axis=1)
    w6 = jnp.concatenate([ws6, wn6 * 0.25], axis=0)
    w7 = jnp.concatenate([ws7, wn7 * 0.25], axis=0)
    wt2 = wt.transpose(0, 2, 3, 1).reshape(wt.shape[0], -1)
    bt4 = jnp.tile(bt, 4).reshape(1, -1)
    bf16 = jnp.bfloat16
    w0, w1, w2, w3, w4, w5, w6, w7, wt2 = (
        a.astype(bf16) for a in (w0, w1, w2, w3, w4, w5, w6, w7, wt2))
    b0r, b1r, b2r, b3r, b4r, b5r, b6r, b7r = (
        b.reshape(1, -1) for b in (b0, b1, b2, b3, b4, b5, b6, b7))

    x = inputs.reshape(BT, HW, C)
    consts = (w0, b0r, w1, b1r, w2, b2r, w3, b3r, w4, b4r, wt2, bt4,
              w5, b5r, w6, b6r, w7, b7r)

    out = pl.pallas_call(
        functools.partial(_body, H=NX, W=NY),
        grid=(BT,),
        in_specs=[_tile_spec(x.shape)] + [_const_spec(a.shape) for a in consts],
        out_specs=[_tile_spec((BT, HW, w7.shape[1]))],
        out_shape=[jax.ShapeDtypeStruct((BT, HW, w7.shape[1]), f32)],
        compiler_params=_PARAMS,
    )(x, *consts)[0]

    return out.reshape(B, T, NX, NY, w7.shape[1])


# speedup vs baseline: 1.1081x; 1.1081x over previous
import functools

import jax
import jax.numpy as jnp
from jax.experimental import pallas as pl
from jax.experimental.pallas import tpu as pltpu

_PARAMS = pltpu.CompilerParams(dimension_semantics=("parallel",))


def _nsum(z2, H, W):
    C = z2.shape[-1]
    F = z2.shape[0] // (H * W)
    x3 = z2.reshape(F, H, W, C)
    up = pltpu.roll(x3, H - 1, axis=1)
    dn = pltpu.roll(x3, 1, axis=1)
    lf = jnp.concatenate([x3[:, :, 1:], x3[:, :, :1]], axis=2)
    rt = pltpu.roll(x3, 1, axis=2)
    return ((up + dn) + (lf + rt)).reshape(F * H * W, C)


def _dot(a, w):
    return jnp.dot(a, w, preferred_element_type=jnp.float32)


def _sage_pre(x2, wm, b, H, W):
    xa = jnp.concatenate([x2, _nsum(x2, H, W)], axis=1)
    return _dot(xa, wm) + b


def _sage_post(x2, wm, b, H, W, co):
    yz = _dot(x2, wm)
    return yz[:, :co] + _nsum(yz[:, co:], H, W) + b


def _body(x_ref, w0, b0, w1, b1, w2, b2, w3, b3, w4, b4, wt, bt,
          w5, b5, w6, b6, w7, b7, o_ref, *, H, W):
    relu = jax.nn.relu
    x = x_ref[0]
    x0 = relu(_sage_pre(x, w0[...], b0[...], H, W))
    h1 = relu(_sage_pre(x0, w1[...], b1[...], H, W))
    h2 = relu(_sage_pre(h1, w2[...], b2[...], H, W))

    C = h2.shape[-1]
    F = h2.shape[0] // (H * W)
    x3 = h2.reshape(F, H, W, C)
    mw = jnp.maximum(x3, pltpu.roll(x3, 1, axis=2))
    mh = jnp.maximum(mw, pltpu.roll(mw, 1, axis=1))
    p = pltpu.einshape("f(ip)(jq)c->pqfijc", mh,
                       i=H // 2, p=2, j=W // 2, q=2)[1, 1]
    p = p.reshape(F * H * W // 4, C)

    h2d, w2d = H // 2, W // 2
    l1 = relu(_sage_pre(p, w3[...], b3[...], h2d, w2d))
    l2 = relu(_sage_pre(l1, w4[...], b4[...], h2d, w2d))

    u = _dot(l2, wt[...]) + bt[...]
    uc = u.shape[-1] // 4
    F = u.shape[0] // (h2d * w2d)
    u6 = u.reshape(F, h2d, w2d, 2, 2, uc)
    u48 = pltpu.einshape("fhwklc->f(hk)(wl)c", u6).reshape(F * H * W, uc)

    cat = jnp.concatenate([h2, u48], axis=1)
    c5 = relu(_sage_post(cat, w5[...], b5[...], H, W, w5.shape[-1] // 2))
    c6 = relu(_sage_pre(c5, w6[...], b6[...], H, W))
    o_ref[0] = _sage_pre(c6, w7[...], b7[...], H, W)


def _const_spec(shape):
    nd = len(shape)
    return pl.BlockSpec(shape, lambda i: (0,) * nd)


def _tile_spec(shape):
    return pl.BlockSpec((1,) + shape[1:], lambda i: (i,) + (0,) * (len(shape) - 1))


def kernel(inputs, ws0, wn0, b0, ws1, wn1, b1, ws2, wn2, b2, ws3, wn3, b3,
           ws4, wn4, b4, ws5, wn5, b5, ws6, wn6, b6, ws7, wn7, b7, wt, bt):
    B, T, NX, NY, C = inputs.shape
    BT = B * T
    HW = NX * NY
    f32 = jnp.float32

    w0 = jnp.concatenate([ws0, wn0 * 0.25], axis=0)
    w1 = jnp.concatenate([ws1, wn1 * 0.25], axis=0)
    w2 = jnp.concatenate([ws2, wn2 * 0.25], axis=0)
    w3 = jnp.concatenate([ws3, wn3 * 0.25], axis=0)
    w4 = jnp.concatenate([ws4, wn4 * 0.25], axis=0)
    w5 = jnp.concatenate([ws5, wn5 * 0.25], axis=1)
    w6 = jnp.concatenate([ws6, wn6 * 0.25], axis=0)
    w7 = jnp.concatenate([ws7, wn7 * 0.25], axis=0)
    wt2 = wt.transpose(0, 2, 3, 1).reshape(wt.shape[0], -1)
    bt4 = jnp.tile(bt, 4).reshape(1, -1)
    b0r, b1r, b2r, b3r, b4r, b5r, b6r, b7r = (
        b.reshape(1, -1) for b in (b0, b1, b2, b3, b4, b5, b6, b7))

    FPB = 2
    x = inputs.reshape(BT // FPB, FPB * HW, C)
    consts = (w0, b0r, w1, b1r, w2, b2r, w3, b3r, w4, b4r, wt2, bt4,
              w5, b5r, w6, b6r, w7, b7r)

    out = pl.pallas_call(
        functools.partial(_body, H=NX, W=NY),
        grid=(BT // FPB,),
        in_specs=[_tile_spec(x.shape)] + [_const_spec(a.shape) for a in consts],
        out_specs=[_tile_spec((BT // FPB, FPB * HW, w7.shape[1]))],
        out_shape=[jax.ShapeDtypeStruct((BT // FPB, FPB * HW, w7.shape[1]), f32)],
        compiler_params=_PARAMS,
    )(x, *consts)[0]

    return out.reshape(B, T, NX, NY, w7.shape[1])
